# baseline (speedup 1.0000x reference)
import jax
import jax.numpy as jnp
from jax.experimental import pallas as pl


from jax.experimental.pallas import tpu as pltpu

S_BLK = 1024


def _bcast_kernel(table_ref, out_ref):
    out_ref[...] = jnp.broadcast_to(table_ref[...][None], out_ref.shape)


def kernel(inputs, table):
    batch, seq = inputs.shape
    n_rows, d_model = table.shape
    grid = (seq // S_BLK,)
    return pl.pallas_call(
        _bcast_kernel,
        grid=grid,
        in_specs=[
            pl.BlockSpec((S_BLK, d_model), lambda i: (i, 0)),
        ],
        out_specs=pl.BlockSpec((batch, S_BLK, d_model), lambda i: (0, i, 0)),
        out_shape=jax.ShapeDtypeStruct((batch, seq, d_model), table.dtype),
        compiler_params=pltpu.CompilerParams(
            dimension_semantics=("arbitrary",),
        ),
    )(table)


# speedup vs baseline: 1.0190x; 1.0190x over previous
import jax
import jax.numpy as jnp
from jax.experimental import pallas as pl
from jax.experimental.pallas import tpu as pltpu

S_BLK = 1024
NBUF = 4


def kernel(inputs, table):
    batch, seq = inputs.shape
    n_rows, d_model = table.shape
    n_blocks = seq // S_BLK

    def body(table_hbm, out_hbm, vmem, in_sems, out_sems):
        i = pl.program_id(0)

        def in_copy(j, slot):
            return pltpu.make_async_copy(
                table_hbm.at[pl.ds(j * S_BLK, S_BLK), :],
                vmem.at[slot],
                in_sems.at[slot],
            )

        def out_copy(j, slot, b):
            return pltpu.make_async_copy(
                vmem.at[slot],
                out_hbm.at[b, pl.ds(j * S_BLK, S_BLK), :],
                out_sems.at[slot, b],
            )

        slot = jax.lax.rem(i, NBUF)

        @pl.when(i == 0)
        def _():
            in_copy(0, 0).start()

        @pl.when(i + 1 < n_blocks)
        def _():
            nslot = jax.lax.rem(i + 1, NBUF)

            @pl.when(i + 1 >= NBUF)
            def _():
                for b in range(batch):
                    out_copy(i + 1 - NBUF, nslot, b).wait()

            in_copy(i + 1, nslot).start()

        in_copy(i, slot).wait()
        for b in range(batch):
            out_copy(i, slot, b).start()

        @pl.when(i == n_blocks - 1)
        def _():
            for j in range(max(0, n_blocks - NBUF), n_blocks):
                for b in range(batch):
                    out_copy(j, j % NBUF, b).wait()

    return pl.pallas_call(
        body,
        grid=(n_blocks,),
        in_specs=[pl.BlockSpec(memory_space=pl.ANY)],
        out_specs=pl.BlockSpec(memory_space=pl.ANY),
        out_shape=jax.ShapeDtypeStruct((batch, seq, d_model), table.dtype),
        scratch_shapes=[
            pltpu.VMEM((NBUF, S_BLK, d_model), table.dtype),
            pltpu.SemaphoreType.DMA((NBUF,)),
            pltpu.SemaphoreType.DMA((NBUF, batch)),
        ],
    )(table)


# speedup vs baseline: 1.0409x; 1.0215x over previous
import jax
import jax.numpy as jnp
from jax.experimental import pallas as pl
from jax.experimental.pallas import tpu as pltpu

S_BLK = 2048
NBUF = 4


def kernel(inputs, table):
    batch, seq = inputs.shape
    n_rows, d_model = table.shape
    n_blocks = seq // S_BLK

    def body(table_hbm, out_hbm, vmem, in_sems, out_sems):
        i = pl.program_id(0)

        def in_copy(j, slot):
            return pltpu.make_async_copy(
                table_hbm.at[pl.ds(j * S_BLK, S_BLK), :],
                vmem.at[slot],
                in_sems.at[slot],
            )

        def out_copy(j, slot, b):
            return pltpu.make_async_copy(
                vmem.at[slot],
                out_hbm.at[b, pl.ds(j * S_BLK, S_BLK), :],
                out_sems.at[slot, b],
            )

        slot = jax.lax.rem(i, NBUF)

        @pl.when(i == 0)
        def _():
            in_copy(0, 0).start()

        @pl.when(i + 1 < n_blocks)
        def _():
            nslot = jax.lax.rem(i + 1, NBUF)

            @pl.when(i + 1 >= NBUF)
            def _():
                for b in range(batch):
                    out_copy(i + 1 - NBUF, nslot, b).wait()

            in_copy(i + 1, nslot).start()

        in_copy(i, slot).wait()
        for b in range(batch):
            out_copy(i, slot, b).start()

        @pl.when(i == n_blocks - 1)
        def _():
            for j in range(max(0, n_blocks - NBUF), n_blocks):
                for b in range(batch):
                    out_copy(j, j % NBUF, b).wait()

    return pl.pallas_call(
        body,
        grid=(n_blocks,),
        in_specs=[pl.BlockSpec(memory_space=pl.ANY)],
        out_specs=pl.BlockSpec(memory_space=pl.ANY),
        out_shape=jax.ShapeDtypeStruct((batch, seq, d_model), table.dtype),
        scratch_shapes=[
            pltpu.VMEM((NBUF, S_BLK, d_model), table.dtype),
            pltpu.SemaphoreType.DMA((NBUF,)),
            pltpu.SemaphoreType.DMA((NBUF, batch)),
        ],
    )(table)


# speedup vs baseline: 1.0546x; 1.0131x over previous
import jax
import jax.numpy as jnp
from jax.experimental import pallas as pl
from jax.experimental.pallas import tpu as pltpu

S_BLK = 4096
NBUF = 2


def kernel(inputs, table):
    batch, seq = inputs.shape
    n_rows, d_model = table.shape
    n_blocks = seq // S_BLK

    def body(table_hbm, out_hbm, vmem, in_sems, out_sems):
        i = pl.program_id(0)

        def in_copy(j, slot):
            return pltpu.make_async_copy(
                table_hbm.at[pl.ds(j * S_BLK, S_BLK), :],
                vmem.at[slot],
                in_sems.at[slot],
            )

        def out_copy(j, slot, b):
            return pltpu.make_async_copy(
                vmem.at[slot],
                out_hbm.at[b, pl.ds(j * S_BLK, S_BLK), :],
                out_sems.at[slot, b],
            )

        slot = jax.lax.rem(i, NBUF)

        @pl.when(i == 0)
        def _():
            in_copy(0, 0).start()

        @pl.when(i + 1 < n_blocks)
        def _():
            nslot = jax.lax.rem(i + 1, NBUF)

            @pl.when(i + 1 >= NBUF)
            def _():
                for b in range(batch):
                    out_copy(i + 1 - NBUF, nslot, b).wait()

            in_copy(i + 1, nslot).start()

        in_copy(i, slot).wait()
        for b in range(batch):
            out_copy(i, slot, b).start()

        @pl.when(i == n_blocks - 1)
        def _():
            for j in range(max(0, n_blocks - NBUF), n_blocks):
                for b in range(batch):
                    out_copy(j, j % NBUF, b).wait()

    return pl.pallas_call(
        body,
        grid=(n_blocks,),
        in_specs=[pl.BlockSpec(memory_space=pl.ANY)],
        out_specs=pl.BlockSpec(memory_space=pl.ANY),
        out_shape=jax.ShapeDtypeStruct((batch, seq, d_model), table.dtype),
        scratch_shapes=[
            pltpu.VMEM((NBUF, S_BLK, d_model), table.dtype),
            pltpu.SemaphoreType.DMA((NBUF,)),
            pltpu.SemaphoreType.DMA((NBUF, batch)),
        ],
    )(table)
